# baseline (speedup 1.0000x reference)
import functools

import jax
import jax.numpy as jnp
from jax import lax
from jax.experimental import pallas as pl
from jax.experimental.pallas import tpu as pltpu
from jax.experimental.pallas import tpu_sc as plsc

N_BUCKETS = 1000000
BIGRAM_DIM = 64
D_MODEL = 1024
B, S = 4, 4096
N = B * S

NC, NS, L = 2, 16, 16
NW = NC * NS
CHUNK = N // NW
NVEC = CHUNK // L
GBATCH = 128
NBATCH = CHUNK // GBATCH

_sc_mesh = plsc.VectorSubcoreMesh(core_axis_name="c", subcore_axis_name="s")


@functools.partial(
    pl.kernel,
    out_type=jax.ShapeDtypeStruct((N, BIGRAM_DIM // 2), jnp.int32),
    mesh=_sc_mesh,
    scratch_types=[
        pltpu.VMEM((CHUNK + L,), jnp.int32),
        pltpu.VMEM((CHUNK + L,), jnp.int32),
        pltpu.SemaphoreType.DMA,
    ],
)
def _sc_hash_gather(ids_hbm, table_hbm, out_hbm, ids_ext, hv, sem_g):
    wid = lax.axis_index("s") * NC + lax.axis_index("c")
    base = wid * CHUNK

    pltpu.sync_copy(ids_hbm.at[pl.ds(base, CHUNK)], ids_ext.at[pl.ds(L, CHUNK)])

    @pl.when(wid != 0)
    def _():
        pltpu.sync_copy(ids_hbm.at[pl.ds(base - L, L)], ids_ext.at[pl.ds(0, L)])

    lane = lax.iota(jnp.int32, L)
    rs = 1 - jnp.clip(base % S, 0, 1)
    keep0 = 1 - rs * jnp.clip(1 - lane, 0, 1)
    for i in range(NVEC):
        cur = ids_ext[pl.ds(L + i * L, L)]
        prv = ids_ext[pl.ds(L - 1 + i * L, L)]
        if i == 0:
            prv = prv * keep0
        h = (prv * 1009 + cur) % N_BUCKETS
        hv[pl.ds(i * L, L)] = h

    def enqueue(t, _):
        h = hv[pl.ds(t, L)][0]
        pltpu.make_async_copy(
            table_hbm.at[pl.ds(h, 1), :],
            out_hbm.at[pl.ds(base + t, 1), :],
            sem_g,
        ).start()
        return 0

    def drain(t, _):
        pltpu.make_async_copy(
            table_hbm.at[pl.ds(0, 1), :],
            out_hbm.at[pl.ds(0, 1), :],
            sem_g,
        ).wait()
        return 0

    for b in range(NBATCH):
        lax.fori_loop(b * GBATCH, (b + 1) * GBATCH, enqueue, 0, unroll=8)
        lax.fori_loop(0, GBATCH, drain, 0, unroll=8)


def _tc_matmul_body(emb_ref, proj_ref, out_ref):
    out_ref[...] = lax.dot_general(
        emb_ref[...],
        proj_ref[...],
        (((1,), (1,)), ((), ())),
        preferred_element_type=jnp.float32,
    )


_ROWS_BLK = 2048
_tc_matmul = pl.pallas_call(
    _tc_matmul_body,
    grid=(N // _ROWS_BLK,),
    in_specs=[
        pl.BlockSpec((_ROWS_BLK, BIGRAM_DIM), lambda i: (i, 0)),
        pl.BlockSpec((D_MODEL, BIGRAM_DIM), lambda i: (0, 0)),
    ],
    out_specs=pl.BlockSpec((_ROWS_BLK, D_MODEL), lambda i: (i, 0)),
    out_shape=jax.ShapeDtypeStruct((N, D_MODEL), jnp.float32),
)


@jax.jit
def kernel(ids, embed_weight, proj_weight):
    ids_flat = ids.reshape(N).astype(jnp.int32)
    lo = lax.bitcast_convert_type(
        embed_weight[:, 0::2].astype(jnp.bfloat16), jnp.uint16
    ).astype(jnp.uint32)
    hi = lax.bitcast_convert_type(
        embed_weight[:, 1::2].astype(jnp.bfloat16), jnp.uint16
    ).astype(jnp.uint32)
    tab_packed = lax.bitcast_convert_type(lo | (hi << 16), jnp.int32)
    emb_packed = _sc_hash_gather(ids_flat, tab_packed)
    ep = lax.bitcast_convert_type(emb_packed, jnp.uint32)
    e_lo = lax.bitcast_convert_type((ep & 0xFFFF).astype(jnp.uint16), jnp.bfloat16)
    e_hi = lax.bitcast_convert_type((ep >> 16).astype(jnp.uint16), jnp.bfloat16)
    emb = jnp.stack([e_lo, e_hi], axis=-1).reshape(N, BIGRAM_DIM)
    out = _tc_matmul(emb, proj_weight.astype(jnp.bfloat16))
    return out.reshape(B, S, D_MODEL)


# speedup vs baseline: 6.1971x; 6.1971x over previous
import functools

import jax
import jax.numpy as jnp
from jax import lax
from jax.experimental import pallas as pl
from jax.experimental.pallas import tpu as pltpu
from jax.experimental.pallas import tpu_sc as plsc

N_BUCKETS = 1000000
BIGRAM_DIM = 64
D_MODEL = 1024
B, S = 4, 4096
N = B * S

NC, NS, L = 2, 16, 16
NW = NC * NS
CHUNK = N // NW
NVEC = CHUNK // L
GBATCH = 128
NBATCH = CHUNK // GBATCH

_sc_mesh = plsc.VectorSubcoreMesh(core_axis_name="c", subcore_axis_name="s")


def _tc_transpose_body(tab_t_ref, out_ref):
    out_ref[...] = tab_t_ref[...].T


_TBLK = 8192
_tc_transpose = pl.pallas_call(
    _tc_transpose_body,
    grid=(pl.cdiv(N_BUCKETS, _TBLK),),
    in_specs=[pl.BlockSpec((BIGRAM_DIM, _TBLK), lambda i: (0, i))],
    out_specs=pl.BlockSpec((_TBLK, BIGRAM_DIM), lambda i: (i, 0)),
    out_shape=jax.ShapeDtypeStruct((N_BUCKETS, BIGRAM_DIM), jnp.float32),
)


@functools.partial(
    pl.kernel,
    out_type=jax.ShapeDtypeStruct((N, BIGRAM_DIM), jnp.float32),
    mesh=_sc_mesh,
    scratch_types=[
        pltpu.VMEM((CHUNK + L,), jnp.int32),
        pltpu.VMEM((CHUNK + L,), jnp.int32),
        pltpu.VMEM((CHUNK, BIGRAM_DIM), jnp.float32),
        pltpu.SemaphoreType.DMA,
    ],
)
def _sc_hash_gather(ids_hbm, table_hbm, out_hbm, ids_ext, hv, rows_v, sem_g):
    wid = lax.axis_index("s") * NC + lax.axis_index("c")
    base = wid * CHUNK

    pltpu.sync_copy(ids_hbm.at[pl.ds(base, CHUNK)], ids_ext.at[pl.ds(L, CHUNK)])

    @pl.when(wid != 0)
    def _():
        pltpu.sync_copy(ids_hbm.at[pl.ds(base - L, L)], ids_ext.at[pl.ds(0, L)])

    lane = lax.iota(jnp.int32, L)
    rs = 1 - jnp.clip(base % S, 0, 1)
    keep0 = 1 - rs * jnp.clip(1 - lane, 0, 1)
    for i in range(NVEC):
        cur = ids_ext[pl.ds(L + i * L, L)]
        prv = ids_ext[pl.ds(L - 1 + i * L, L)]
        if i == 0:
            prv = prv * keep0
        h = (prv * 1009 + cur) % N_BUCKETS
        hv[pl.ds(i * L, L)] = h

    def enqueue(t, _):
        h = hv[pl.ds(t, L)][0]
        pltpu.make_async_copy(
            table_hbm.at[pl.ds(h, 1), :],
            rows_v.at[pl.ds(t, 1), :],
            sem_g,
        ).start()
        return 0

    def drain(t, _):
        pltpu.make_async_copy(
            table_hbm.at[pl.ds(0, 1), :],
            rows_v.at[pl.ds(0, 1), :],
            sem_g,
        ).wait()
        return 0

    for b in range(NBATCH):
        lax.fori_loop(b * GBATCH, (b + 1) * GBATCH, enqueue, 0, unroll=8)
        lax.fori_loop(0, GBATCH, drain, 0, unroll=8)

    pltpu.sync_copy(rows_v, out_hbm.at[pl.ds(base, CHUNK)])


def _tc_matmul_body(emb_ref, proj_ref, out_ref):
    out_ref[...] = lax.dot_general(
        emb_ref[...],
        proj_ref[...],
        (((1,), (1,)), ((), ())),
        preferred_element_type=jnp.float32,
    )


_ROWS_BLK = 2048
_tc_matmul = pl.pallas_call(
    _tc_matmul_body,
    grid=(N // _ROWS_BLK,),
    in_specs=[
        pl.BlockSpec((_ROWS_BLK, BIGRAM_DIM), lambda i: (i, 0)),
        pl.BlockSpec((D_MODEL, BIGRAM_DIM), lambda i: (0, 0)),
    ],
    out_specs=pl.BlockSpec((_ROWS_BLK, D_MODEL), lambda i: (i, 0)),
    out_shape=jax.ShapeDtypeStruct((N, D_MODEL), jnp.float32),
)


@jax.jit
def kernel(ids, embed_weight, proj_weight):
    ids_flat = ids.reshape(N).astype(jnp.int32)
    table_rm = _tc_transpose(embed_weight.T)
    emb = _sc_hash_gather(ids_flat, table_rm)
    out = _tc_matmul(emb, proj_weight)
    return out.reshape(B, S, D_MODEL)
